# baseline (speedup 1.0000x reference)
import functools

import jax
import jax.numpy as jnp
from jax.experimental import pallas as pl


def _ecc_block_kernel(x_ref, proto_ref, out_ref, *, num_classes):
    xb = x_ref[0]
    proto = proto_ref[...]
    p_sq = jnp.sum(proto * proto, axis=1, keepdims=True)
    dots = jax.lax.dot_general(
        proto, xb, (((1,), (0,)), ((), ())),
        preferred_element_type=jnp.float32)
    sq = p_sq - 2.0 * dots
    kp, t = sq.shape
    sqm = jnp.max(sq.reshape(num_classes, kp // num_classes, t), axis=1)
    x_sq = jnp.sum(xb * xb, axis=0, keepdims=True)
    out_ref[0] = jnp.sqrt(jnp.maximum(sqm + x_sq, 0.0))


def kernel(x, gt, prototype):
    del gt
    B, C, H, W = x.shape
    K, P, _ = prototype.shape
    KP = K * P
    HW = H * W
    T = 2304

    xr = x.reshape(B, C, HW)
    proto = prototype.reshape(KP, C)

    out = pl.pallas_call(
        functools.partial(_ecc_block_kernel, num_classes=K),
        grid=(B, HW // T),
        in_specs=[
            pl.BlockSpec((1, C, T), lambda b, t: (b, 0, t)),
            pl.BlockSpec((KP, C), lambda b, t: (0, 0)),
        ],
        out_specs=pl.BlockSpec((1, K, T), lambda b, t: (b, 0, t)),
        out_shape=jax.ShapeDtypeStruct((B, K, HW), jnp.float32),
    )(xr, proto)
    return out.reshape(B, K, H, W)


# speedup vs baseline: 1.0302x; 1.0302x over previous
import functools

import jax
import jax.numpy as jnp
from jax.experimental import pallas as pl


def _ecc_block_kernel(x_ref, proto_ref, out_ref, *, num_classes):
    xb = x_ref[0]
    proto = proto_ref[...]
    p_sq = jnp.sum(proto * proto, axis=1, keepdims=True)
    dots = jax.lax.dot_general(
        proto, xb, (((1,), (0,)), ((), ())),
        preferred_element_type=jnp.float32)
    sq = p_sq - 2.0 * dots
    kp, t = sq.shape
    sqm = jnp.max(sq.reshape(num_classes, kp // num_classes, t), axis=1)
    x_sq = jnp.sum(xb * xb, axis=0, keepdims=True)
    out_ref[0] = jnp.sqrt(jnp.maximum(sqm + x_sq, 0.0))


def kernel(x, gt, prototype):
    del gt
    B, C, H, W = x.shape
    K, P, _ = prototype.shape
    KP = K * P
    HW = H * W
    T = 9216

    xr = x.reshape(B, C, HW)
    proto = prototype.reshape(KP, C)

    out = pl.pallas_call(
        functools.partial(_ecc_block_kernel, num_classes=K),
        grid=(B, HW // T),
        in_specs=[
            pl.BlockSpec((1, C, T), lambda b, t: (b, 0, t)),
            pl.BlockSpec((KP, C), lambda b, t: (0, 0)),
        ],
        out_specs=pl.BlockSpec((1, K, T), lambda b, t: (b, 0, t)),
        out_shape=jax.ShapeDtypeStruct((B, K, HW), jnp.float32),
    )(xr, proto)
    return out.reshape(B, K, H, W)
